# baseline (speedup 1.0000x reference)
import jax
import jax.numpy as jnp
from jax.experimental import pallas as pl
from jax.experimental.pallas import tpu as pltpu

BATCH = 2048
EMB = 256
MEM = 16384
BLK = 128
GRID = BATCH // BLK
WIN = 9 * BLK
EXT = BATCH + WIN - BLK
AUG = EMB + 2
NB_MEM = 8
MEM_BLK = MEM // NB_MEM


def _mem_copies(x_ref, lrow_ref, emem_ref, lmem_ref, zf_ref, zi_ref, sem):
    cps = [pltpu.make_async_copy(x_ref, emem_ref.at[pl.ds(0, MEM_BLK), :],
                                 sem),
           pltpu.make_async_copy(lrow_ref, lmem_ref.at[pl.ds(0, 1), :], sem)]
    for k in range(1, NB_MEM):
        cps.append(pltpu.make_async_copy(
            zf_ref, emem_ref.at[pl.ds(k * MEM_BLK, MEM_BLK), :], sem))
        cps.append(pltpu.make_async_copy(
            zi_ref, lmem_ref.at[pl.ds(k, 1), :], sem))
    return cps


def _cbm_kernel(x_ref, lrow_ref, lcol_ref, loss_ref,
                emem_ref, lmem_ref, acc_ref, tile_ref, xb_ref,
                lb_ref, lhs_ref, zf_ref, zi_ref, sem):
    i = pl.program_id(0)

    @pl.when(i == 0)
    def _():
        x = x_ref[...]
        zf_ref[...] = jnp.zeros_like(zf_ref)
        zi_ref[...] = jnp.zeros_like(zi_ref)
        for cp in _mem_copies(x_ref, lrow_ref, emem_ref, lmem_ref,
                              zf_ref, zi_ref, sem):
            cp.start()
        xb = x.astype(jnp.bfloat16)
        sqb = jnp.sum(x * x, axis=1, keepdims=True).astype(jnp.bfloat16)
        xb_ref[0:BATCH, 0:EMB] = xb
        xb_ref[BATCH:EXT, 0:EMB] = xb[0:EXT - BATCH, :]
        xb_ref[0:BATCH, EMB:EMB + 1] = jnp.ones((BATCH, 1), jnp.bfloat16)
        xb_ref[BATCH:EXT, EMB:EMB + 1] = jnp.ones((EXT - BATCH, 1),
                                                  jnp.bfloat16)
        xb_ref[0:BATCH, EMB + 1:AUG] = sqb
        xb_ref[BATCH:EXT, EMB + 1:AUG] = sqb[0:EXT - BATCH, :]
        lb = lrow_ref[...].astype(jnp.bfloat16)
        lb_ref[0:1, 0:BATCH] = lb
        lb_ref[0:1, BATCH:EXT] = lb[0:1, 0:EXT - BATCH]
        acc_ref[...] = jnp.zeros_like(acc_ref)

    xib = xb_ref[pl.ds(i * BLK, BLK), 0:EMB]
    sqib = xb_ref[pl.ds(i * BLK, BLK), EMB + 1:AUG]
    lhs_ref[0:BLK, 0:EMB] = xib * jnp.bfloat16(-2.0)
    lhs_ref[0:BLK, EMB:EMB + 1] = sqib
    lhs_ref[0:BLK, EMB + 1:AUG] = jnp.ones((BLK, 1), jnp.bfloat16)
    xwa = xb_ref[pl.ds(i * BLK, WIN), :]
    li = lcol_ref[pl.ds(i * BLK, BLK), :]
    lw = lb_ref[0:1, pl.ds(i * BLK, WIN)]

    d2 = jax.lax.dot_general(lhs_ref[...], xwa, (((1,), (1,)), ((), ())),
                             preferred_element_type=jnp.float32)
    dmat_b = jnp.sqrt(jnp.maximum(d2, 1e-12)).astype(jnp.bfloat16)

    match = li == lw
    relu_b = jnp.maximum(jnp.bfloat16(1.0) - dmat_b, jnp.bfloat16(0))
    zero_b = jnp.zeros_like(dmat_b)
    tile_ref[0:BLK, :] = jnp.where(match, dmat_b, zero_b)
    tile_ref[BLK:2 * BLK, :] = jnp.where(match, zero_b, relu_b)
    tile_ref[2 * BLK:3 * BLK, :] = jnp.where(match, jnp.ones_like(dmat_b),
                                             zero_b)
    r3 = jax.lax.broadcasted_iota(jnp.int32, (3, 3 * BLK), 0)
    k3 = jax.lax.broadcasted_iota(jnp.int32, (3, 3 * BLK), 1)
    sel = ((k3 >= r3 * BLK) & (k3 < (r3 + 1) * BLK)).astype(jnp.bfloat16)
    red = jax.lax.dot_general(sel, tile_ref[...], (((1,), (0,)), ((), ())),
                              preferred_element_type=jnp.float32)
    acc_ref[0:3, :] += red

    @pl.when(i == GRID - 1)
    def _():
        c = jax.lax.broadcasted_iota(jnp.int32, (1, WIN), 1)
        w = 1.0 + ((c >= BLK) &
                   (c < (WIN // BLK - 1) * BLK)).astype(jnp.float32)
        n = jnp.float32(BATCH)
        a = jnp.sum(acc_ref[0:1, :] * w)
        b = jnp.sum(acc_ref[1:2, :] * w)
        cnt = jnp.sum(acc_ref[2:3, :] * w)
        loss = a / (cnt - n) + b / (n * n - cnt)
        loss_ref[...] = jnp.full((1, 1), loss, jnp.float32)
        for cp in _mem_copies(x_ref, lrow_ref, emem_ref, lmem_ref,
                              zf_ref, zi_ref, sem):
            cp.wait()


def kernel(embeddings, labels, embedding_memory, label_memory):
    labels = labels.astype(jnp.int32)
    lab_row = labels.reshape(1, BATCH)
    lab_col = labels.reshape(BATCH, 1).astype(jnp.bfloat16)

    loss, emem, lmem = pl.pallas_call(
        _cbm_kernel,
        grid=(GRID,),
        in_specs=[
            pl.BlockSpec((BATCH, EMB), lambda i: (0, 0)),
            pl.BlockSpec((1, BATCH), lambda i: (0, 0)),
            pl.BlockSpec((BATCH, 1), lambda i: (0, 0)),
        ],
        out_specs=(
            pl.BlockSpec((1, 1), lambda i: (0, 0)),
            pl.BlockSpec(memory_space=pl.ANY),
            pl.BlockSpec(memory_space=pl.ANY),
        ),
        out_shape=(
            jax.ShapeDtypeStruct((1, 1), jnp.float32),
            jax.ShapeDtypeStruct((MEM, EMB), jnp.float32),
            jax.ShapeDtypeStruct((NB_MEM, MEM_BLK), jnp.int32),
        ),
        scratch_shapes=[
            pltpu.VMEM((4, WIN), jnp.float32),
            pltpu.VMEM((3 * BLK, WIN), jnp.bfloat16),
            pltpu.VMEM((EXT, AUG), jnp.bfloat16),
            pltpu.VMEM((1, EXT), jnp.bfloat16),
            pltpu.VMEM((BLK, AUG), jnp.bfloat16),
            pltpu.VMEM((MEM_BLK, EMB), jnp.float32),
            pltpu.VMEM((1, MEM_BLK), jnp.int32),
            pltpu.SemaphoreType.DMA,
        ],
        compiler_params=pltpu.CompilerParams(
            dimension_semantics=("arbitrary",)),
    )(embeddings, lab_row, lab_col)
    return loss.reshape(()), emem, lmem.reshape(MEM)


# speedup vs baseline: 1.1606x; 1.1606x over previous
import jax
import jax.numpy as jnp
from jax.experimental import pallas as pl
from jax.experimental.pallas import tpu as pltpu

BATCH = 2048
EMB = 256
MEM = 16384
BLK = 256
GRID = BATCH // BLK
WIN = 5 * BLK
EXT = BATCH + WIN - BLK
AUG = EMB + 2
MEM_BLK = MEM // GRID


def _mem_copies(x_ref, lrow_ref, emem_ref, lmem_ref, zf_ref, zi_ref, sem):
    cps = [pltpu.make_async_copy(x_ref, emem_ref.at[pl.ds(0, MEM_BLK), :],
                                 sem),
           pltpu.make_async_copy(lrow_ref, lmem_ref.at[pl.ds(0, 1), :], sem)]
    for k in range(1, GRID):
        cps.append(pltpu.make_async_copy(
            zf_ref, emem_ref.at[pl.ds(k * MEM_BLK, MEM_BLK), :], sem))
        cps.append(pltpu.make_async_copy(
            zi_ref, lmem_ref.at[pl.ds(k, 1), :], sem))
    return cps


def _cbm_kernel(x_ref, lrow_ref, lcol_ref, loss_ref,
                emem_ref, lmem_ref, acc_ref, tile_ref, xb_ref,
                lb_ref, lhs_ref, zf_ref, zi_ref, sem):
    i = pl.program_id(0)

    @pl.when(i == 0)
    def _():
        x = x_ref[...]
        zf_ref[...] = jnp.zeros_like(zf_ref)
        zi_ref[...] = jnp.zeros_like(zi_ref)
        for cp in _mem_copies(x_ref, lrow_ref, emem_ref, lmem_ref,
                              zf_ref, zi_ref, sem):
            cp.start()
        xb = x.astype(jnp.bfloat16)
        sqb = jnp.sum(x * x, axis=1, keepdims=True).astype(jnp.bfloat16)
        xb_ref[0:BATCH, 0:EMB] = xb
        xb_ref[BATCH:EXT, 0:EMB] = xb[0:EXT - BATCH, :]
        xb_ref[0:BATCH, EMB:EMB + 1] = jnp.ones((BATCH, 1), jnp.bfloat16)
        xb_ref[BATCH:EXT, EMB:EMB + 1] = jnp.ones((EXT - BATCH, 1),
                                                  jnp.bfloat16)
        xb_ref[0:BATCH, EMB + 1:AUG] = sqb
        xb_ref[BATCH:EXT, EMB + 1:AUG] = sqb[0:EXT - BATCH, :]
        lb = lrow_ref[...].astype(jnp.bfloat16)
        lb_ref[0:1, 0:BATCH] = lb
        lb_ref[0:1, BATCH:EXT] = lb[0:1, 0:EXT - BATCH]
        acc_ref[...] = jnp.zeros_like(acc_ref)

    xib = xb_ref[pl.ds(i * BLK, BLK), 0:EMB]
    sqib = xb_ref[pl.ds(i * BLK, BLK), EMB + 1:AUG]
    lhs_ref[0:BLK, 0:EMB] = xib * jnp.bfloat16(-2.0)
    lhs_ref[0:BLK, EMB:EMB + 1] = sqib
    lhs_ref[0:BLK, EMB + 1:AUG] = jnp.ones((BLK, 1), jnp.bfloat16)
    xwa = xb_ref[pl.ds(i * BLK, WIN), :]
    li = lcol_ref[pl.ds(i * BLK, BLK), :]
    lw = lb_ref[0:1, pl.ds(i * BLK, WIN)]

    d2 = jax.lax.dot_general(lhs_ref[...], xwa, (((1,), (1,)), ((), ())),
                             preferred_element_type=jnp.float32)
    dmat_b = jnp.sqrt(jnp.maximum(d2, 1e-12)).astype(jnp.bfloat16)

    match = li == lw
    relu_b = jnp.maximum(jnp.bfloat16(1.0) - dmat_b, jnp.bfloat16(0))
    zero_b = jnp.zeros_like(dmat_b)
    tile_ref[0:BLK, :] = jnp.where(match, dmat_b, zero_b)
    tile_ref[BLK:2 * BLK, :] = jnp.where(match, zero_b, relu_b)
    tile_ref[2 * BLK:3 * BLK, :] = jnp.where(match, jnp.ones_like(dmat_b),
                                             zero_b)
    r3 = jax.lax.broadcasted_iota(jnp.int32, (3, 3 * BLK), 0)
    k3 = jax.lax.broadcasted_iota(jnp.int32, (3, 3 * BLK), 1)
    sel = ((k3 >= r3 * BLK) & (k3 < (r3 + 1) * BLK)).astype(jnp.bfloat16)
    red = jax.lax.dot_general(sel, tile_ref[...], (((1,), (0,)), ((), ())),
                              preferred_element_type=jnp.float32)
    acc_ref[0:3, :] += red

    @pl.when(i == GRID - 1)
    def _():
        c = jax.lax.broadcasted_iota(jnp.int32, (1, WIN), 1)
        w = 1.0 + ((c >= BLK) & (c < 4 * BLK)).astype(jnp.float32)
        n = jnp.float32(BATCH)
        a = jnp.sum(acc_ref[0:1, :] * w)
        b = jnp.sum(acc_ref[1:2, :] * w)
        cnt = jnp.sum(acc_ref[2:3, :] * w)
        loss = a / (cnt - n) + b / (n * n - cnt)
        loss_ref[...] = jnp.full((1, 1), loss, jnp.float32)
        for cp in _mem_copies(x_ref, lrow_ref, emem_ref, lmem_ref,
                              zf_ref, zi_ref, sem):
            cp.wait()


def kernel(embeddings, labels, embedding_memory, label_memory):
    labels = labels.astype(jnp.int32)
    lab_row = labels.reshape(1, BATCH)
    lab_col = labels.reshape(BATCH, 1).astype(jnp.bfloat16)

    loss, emem, lmem = pl.pallas_call(
        _cbm_kernel,
        grid=(GRID,),
        in_specs=[
            pl.BlockSpec((BATCH, EMB), lambda i: (0, 0)),
            pl.BlockSpec((1, BATCH), lambda i: (0, 0)),
            pl.BlockSpec((BATCH, 1), lambda i: (0, 0)),
        ],
        out_specs=(
            pl.BlockSpec((1, 1), lambda i: (0, 0)),
            pl.BlockSpec(memory_space=pl.ANY),
            pl.BlockSpec(memory_space=pl.ANY),
        ),
        out_shape=(
            jax.ShapeDtypeStruct((1, 1), jnp.float32),
            jax.ShapeDtypeStruct((MEM, EMB), jnp.float32),
            jax.ShapeDtypeStruct((GRID, MEM_BLK), jnp.int32),
        ),
        scratch_shapes=[
            pltpu.VMEM((4, WIN), jnp.float32),
            pltpu.VMEM((3 * BLK, WIN), jnp.bfloat16),
            pltpu.VMEM((EXT, AUG), jnp.bfloat16),
            pltpu.VMEM((1, EXT), jnp.bfloat16),
            pltpu.VMEM((BLK, AUG), jnp.bfloat16),
            pltpu.VMEM((MEM_BLK, EMB), jnp.float32),
            pltpu.VMEM((1, MEM_BLK), jnp.int32),
            pltpu.SemaphoreType.DMA,
        ],
        compiler_params=pltpu.CompilerParams(
            dimension_semantics=("arbitrary",)),
    )(embeddings, lab_row, lab_col)
    return loss.reshape(()), emem, lmem.reshape(MEM)
